# Initial kernel scaffold; baseline (speedup 1.0000x reference)
import jax
import jax.numpy as jnp
from jax.experimental import pallas as pl


def kernel(features, edges, w0_0, b0_0, w1_0, b1_0, g_0, be_0, w0_1, b0_1, w1_1, b1_1, g_1, be_1, w0_2, b0_2, w1_2, b1_2, g_2, be_2):
    raise NotImplementedError("write your pallas kernel here")


# speedup vs baseline: 4.4863x; 4.4863x over previous
import functools

import jax
import jax.numpy as jnp
from jax import lax
from jax.experimental import pallas as pl
from jax.experimental.pallas import tpu as pltpu
from jax.experimental.pallas import tpu_sc as plsc

N = 10000
E = 320000
D = 128
EPS = 1e-5

NB = 10
BR = N // NB
ER = (2 * E) // D
NW = 32
RPS = N // 16

_mesh = plsc.VectorSubcoreMesh(core_axis_name="c", subcore_axis_name="s")


def _sc_scatter(with_deg):
    out_type = [jax.ShapeDtypeStruct((NW, RPS, D), jnp.float32)]
    scratch = [
        pltpu.VMEM_SHARED((N, D), jnp.float32),
        pltpu.VMEM((D,), jnp.int32),
        pltpu.VMEM((D,), jnp.int32),
        pltpu.VMEM((D, D), jnp.float32),
        pltpu.SemaphoreType.DMA,
    ]
    if with_deg:
        out_type.append(jax.ShapeDtypeStruct((NW, RPS, 16), jnp.float32))
        scratch += [
            pltpu.VMEM_SHARED((N, 16), jnp.float32),
            pltpu.VMEM((D, 16), jnp.float32),
        ]

    def body(vw1, srcs, dsts, zeros, zeros16, ones_in, part, *rest):
        if with_deg:
            degpart, acc_sh, src_v, dst_v, rows_v, sem, deg_sh, ones_v = rest
        else:
            acc_sh, src_v, dst_v, rows_v, sem = rest
        cid = lax.axis_index("c")
        sid = lax.axis_index("s")
        w = cid * 16 + sid
        pltpu.sync_copy(zeros, acc_sh.at[pl.ds(sid * RPS, RPS)])
        if with_deg:
            pltpu.sync_copy(zeros16, deg_sh.at[pl.ds(sid * RPS, RPS)])
            pltpu.sync_copy(ones_in, ones_v)
        plsc.subcore_barrier()

        nt = 156 + jnp.where(w < ER - 156 * NW, 1, 0)

        def step(t, carry):
            r = w + NW * t
            pltpu.sync_copy(srcs.at[r], src_v)
            pltpu.sync_copy(dsts.at[r], dst_v)
            pltpu.async_copy(vw1.at[src_v], rows_v, sem).wait()
            pltpu.sync_copy(rows_v, acc_sh.at[dst_v], add=True)
            if with_deg:
                pltpu.sync_copy(ones_v, deg_sh.at[dst_v], add=True)
            return carry

        lax.fori_loop(0, nt, step, 0)
        plsc.subcore_barrier()
        pltpu.sync_copy(acc_sh.at[pl.ds(sid * RPS, RPS)], part.at[w])
        if with_deg:
            pltpu.sync_copy(deg_sh.at[pl.ds(sid * RPS, RPS)], degpart.at[w])

    return pl.kernel(body, out_type=out_type, mesh=_mesh, scratch_types=scratch,
                     compiler_params=pltpu.CompilerParams(use_tc_tiling_on_sc=False))


_sc_scatter_deg = _sc_scatter(True)
_sc_scatter_nodeg = _sc_scatter(False)


def _mm2_body(x_ref, w0_ref, b0_ref, w1_ref, b1_ref, o0_ref, o1_ref):
    x = x_ref[...]
    o0_ref[...] = jnp.dot(x, w0_ref[...], preferred_element_type=jnp.float32) + b0_ref[...]
    o1_ref[...] = jnp.dot(x, w1_ref[...], preferred_element_type=jnp.float32) + b1_ref[...]


def _mm2(x, w0, b0, w1, b1):
    blk = pl.BlockSpec((BR, D), lambda i: (i, 0))
    wspec = pl.BlockSpec((D, D), lambda i: (0, 0))
    bspec = pl.BlockSpec((1, D), lambda i: (0, 0))
    return pl.pallas_call(
        _mm2_body,
        grid=(NB,),
        in_specs=[blk, wspec, bspec, wspec, bspec],
        out_specs=[blk, blk],
        out_shape=[jax.ShapeDtypeStruct((N, D), jnp.float32)] * 2,
    )(x, w0, b0.reshape(1, D), w1, b1.reshape(1, D))


def _stats_body(vw0_ref, p_ref, degp_ref, t_ref, sums_ref):
    i = pl.program_id(0)
    deg = degp_ref[0, :, 0] + degp_ref[1, :, 0]
    dinv = 1.0 / (1.0 + deg)
    t = (vw0_ref[...] + p_ref[0] + p_ref[1]) * dinv[:, None]
    t_ref[...] = t
    s = jnp.sum(t, axis=0)
    s2 = jnp.sum(t * t, axis=0)
    upd = jnp.concatenate(
        [s[None, :], s2[None, :], jnp.zeros((6, D), jnp.float32)], axis=0)

    @pl.when(i == 0)
    def _():
        sums_ref[...] = upd

    @pl.when(i > 0)
    def _():
        sums_ref[...] = sums_ref[...] + upd


def _stats(vw0, part, degpart):
    return pl.pallas_call(
        _stats_body,
        grid=(NB,),
        in_specs=[
            pl.BlockSpec((BR, D), lambda i: (i, 0)),
            pl.BlockSpec((2, BR, D), lambda i: (0, i, 0)),
            pl.BlockSpec((2, BR, 16), lambda i: (0, i, 0)),
        ],
        out_specs=[
            pl.BlockSpec((BR, D), lambda i: (i, 0)),
            pl.BlockSpec((8, D), lambda i: (0, 0)),
        ],
        out_shape=[
            jax.ShapeDtypeStruct((N, D), jnp.float32),
            jax.ShapeDtypeStruct((8, D), jnp.float32),
        ],
    )(vw0, part, degpart)


def _apply_factory(with_res):
    def body(*refs):
        if with_res:
            t_ref, sums_ref, g_ref, be_ref, res_ref, o_ref = refs
        else:
            t_ref, sums_ref, g_ref, be_ref, o_ref = refs
        m = sums_ref[0, :] / N
        v = sums_ref[1, :] / N - m * m
        scale = g_ref[0, :] * lax.rsqrt(v + EPS)
        y = (t_ref[...] - m[None, :]) * scale[None, :] + be_ref[0, :][None, :]
        if with_res:
            y = y + res_ref[...]
        o_ref[...] = jnp.maximum(y, 0.0)

    blk = pl.BlockSpec((BR, D), lambda i: (i, 0))
    row = pl.BlockSpec((1, D), lambda i: (0, 0))
    srow = pl.BlockSpec((8, D), lambda i: (0, 0))
    in_specs = [blk, srow, row, row] + ([blk] if with_res else [])
    return pl.pallas_call(
        body,
        grid=(NB,),
        in_specs=in_specs,
        out_specs=blk,
        out_shape=jax.ShapeDtypeStruct((N, D), jnp.float32),
    )


_apply_res = _apply_factory(True)
_apply_nores = _apply_factory(False)


def kernel(features, edges, w0_0, b0_0, w1_0, b1_0, g_0, be_0,
           w0_1, b0_1, w1_1, b1_1, g_1, be_1,
           w0_2, b0_2, w1_2, b1_2, g_2, be_2):
    srcs = jnp.concatenate([edges[:, 1], edges[:, 0]]).reshape(ER, D)
    dsts = jnp.concatenate([edges[:, 0], edges[:, 1]]).reshape(ER, D)
    zeros = jnp.zeros((RPS, D), jnp.float32)
    zeros16 = jnp.zeros((RPS, 16), jnp.float32)
    ones16 = jnp.ones((D, 16), jnp.float32)

    x = features
    degpart = None
    params = [(w0_0, b0_0, w1_0, b1_0, g_0, be_0),
              (w0_1, b0_1, w1_1, b1_1, g_1, be_1),
              (w0_2, b0_2, w1_2, b1_2, g_2, be_2)]
    for li, (w0, b0, w1, b1, g, be) in enumerate(params):
        vw0, vw1 = _mm2(x, w0, b0, w1, b1)
        if li == 0:
            part, degpart = _sc_scatter_deg(vw1, srcs, dsts, zeros, zeros16, ones16)
            degpart = degpart.reshape(2, N, 16)
        else:
            (part,) = _sc_scatter_nodeg(vw1, srcs, dsts, zeros, zeros16, ones16)
        t, sums = _stats(vw0, part.reshape(2, N, D), degpart)
        if li == 2:
            x = _apply_res(t, sums, g.reshape(1, D), be.reshape(1, D), features)
        else:
            x = _apply_nores(t, sums, g.reshape(1, D), be.reshape(1, D))
    return x
